# baseline (speedup 1.0000x reference)
import functools

import jax
import jax.numpy as jnp
from jax import lax
from jax.experimental import pallas as pl
from jax.experimental.pallas import tpu as pltpu

NUM_EXPERTS = 64
D_MODEL = 2048
TBLK = 2048


def _router_kernel(x_ref, w_ref, b_ref, mask_ref, loss_ref, acc_ref, *, nsteps, total_tokens):
    i = pl.program_id(0)

    @pl.when(i == 0)
    def _init():
        acc_ref[...] = jnp.zeros_like(acc_ref)

    x = x_ref[...]
    w = w_ref[...]
    logits = lax.dot_general(
        x, w, (((1,), (1,)), ((), ())),
        preferred_element_type=jnp.float32,
    ) + b_ref[...]

    col = lax.broadcasted_iota(jnp.int32, logits.shape, 1)
    mx = jnp.max(logits, axis=1, keepdims=True)
    idx = jnp.min(jnp.where(logits == mx, col, NUM_EXPERTS), axis=1, keepdims=True)
    mask = (col == idx).astype(jnp.float32)
    mask_ref[...] = mask

    e = jnp.exp(logits - mx)
    probs = e / jnp.sum(e, axis=1, keepdims=True)

    acc_ref[0:1, :] += jnp.sum(mask, axis=0, keepdims=True)
    acc_ref[1:2, :] += jnp.sum(probs, axis=0, keepdims=True)

    @pl.when(i == nsteps - 1)
    def _finish():
        counts = acc_ref[0:1, :]
        psum = acc_ref[1:2, :]
        scale = NUM_EXPERTS / (total_tokens * total_tokens)
        loss_ref[...] = jnp.sum(counts * psum, keepdims=True).reshape(1, 1) * scale


@jax.jit
def kernel(x, W, b):
    B, S, D = x.shape
    T = B * S
    E = W.shape[0]
    xf = x.reshape(T, D)
    nsteps = T // TBLK

    mask, loss = pl.pallas_call(
        functools.partial(_router_kernel, nsteps=nsteps, total_tokens=T),
        grid=(nsteps,),
        in_specs=[
            pl.BlockSpec((TBLK, D), lambda i: (i, 0)),
            pl.BlockSpec((E, D), lambda i: (0, 0)),
            pl.BlockSpec((1, E), lambda i: (0, 0)),
        ],
        out_specs=[
            pl.BlockSpec((TBLK, E), lambda i: (i, 0)),
            pl.BlockSpec((1, 1), lambda i: (0, 0)),
        ],
        out_shape=[
            jax.ShapeDtypeStruct((T, E), jnp.float32),
            jax.ShapeDtypeStruct((1, 1), jnp.float32),
        ],
        scratch_shapes=[pltpu.VMEM((2, E), jnp.float32)],
    )(xf, W, b.reshape(1, E))

    return mask.reshape(B, S, E), loss[0, 0]
